# baseline (device time: 42963 ns/iter reference)
import jax
import jax.numpy as jnp
from jax import lax
from jax.experimental import pallas as pl
from jax.experimental.pallas import tpu as pltpu

N_DEV = 8
B = 2
SQ = 256
DM = 512
DH = 64
HQ_LOC = 4
HQ = N_DEV * HQ_LOC
SKV_LOC = 256
BLK = 64
BF16 = jnp.bfloat16
SC = SQ // N_DEV

MESH = pl.DeviceIdType.MESH


def kernel(x, Wq, K_ext, V_ext, Wo):
    def body(x_ref, wq_ref, k_ref, v_ref, wo_ref, out_ref,
             kf32, vf32, kstage, vstage, kbuf, vbuf, rbufs,
             part, rsbuf, agstage, agbuf,
             copy_sems, ksend_sems, vsend_sems, krecv_sem, vrecv_sem,
             relay_recv_sems, fwd_send_sems,
             rssend_sems, rsrecv_sems, agsend_sems, agrecv_sems):
        me = lax.axis_index("i")

        def slice_rdma(stage, d, dst, send_sem, recv_sem, target):
            return pltpu.make_async_remote_copy(
                src_ref=stage.at[:, pl.ds(HQ_LOC * DH * d, HQ_LOC * DH), :],
                dst_ref=dst, send_sem=send_sem, recv_sem=recv_sem,
                device_id=(target,), device_id_type=MESH)

        def kdir(d):
            return slice_rdma(kstage, d, kbuf, ksend_sems.at[d], krecv_sem, d)

        def vdir(d):
            return slice_rdma(vstage, d, vbuf, vsend_sems.at[d], vrecv_sem, d)

        def fwd(slot, dst, recv_sem, target):
            return pltpu.make_async_remote_copy(
                src_ref=rbufs.at[slot], dst_ref=dst,
                send_sem=fwd_send_sems.at[slot], recv_sem=recv_sem,
                device_id=(target,), device_id_type=MESH)

        def relay_wait(slot):
            pltpu.make_async_remote_copy(
                src_ref=rbufs.at[slot], dst_ref=rbufs.at[slot],
                send_sem=fwd_send_sems.at[slot],
                recv_sem=relay_recv_sems.at[slot],
                device_id=(0,), device_id_type=MESH).wait_recv()

        def ph1_bcast_issue():
            @pl.when(me == 0)
            def _():
                ck = pltpu.make_async_copy(k_ref, kf32, copy_sems.at[0])
                cv = pltpu.make_async_copy(v_ref, vf32, copy_sems.at[1])
                ck.start()
                cv.start()
                ck.wait()
                cv.wait()
                kstage[...] = kf32[...].astype(BF16)
                vstage[...] = vf32[...].astype(BF16)

                slice_rdma(kstage, 5, rbufs.at[0], ksend_sems.at[5],
                           relay_recv_sems.at[0], 4).start()
                slice_rdma(vstage, 5, rbufs.at[1], vsend_sems.at[5],
                           relay_recv_sems.at[1], 4).start()
                slice_rdma(vstage, 7, rbufs.at[2], vsend_sems.at[7],
                           relay_recv_sems.at[2], 4).start()
                slice_rdma(kstage, 6, rbufs.at[0], ksend_sems.at[6],
                           relay_recv_sems.at[0], 3).start()
                slice_rdma(vstage, 6, rbufs.at[1], vsend_sems.at[6],
                           relay_recv_sems.at[1], 3).start()
                kdir(7).start()
                kdir(2).start()
                kdir(1).start()

                kbuf[...] = kstage[:, 0:HQ_LOC * DH, :]
                vbuf[...] = vstage[:, 0:HQ_LOC * DH, :]

                kdir(2).wait_send()
                kdir(1).wait_send()
                vdir(2).start()
                vdir(1).start()

                slice_rdma(kstage, 5, rbufs.at[0], ksend_sems.at[5],
                           relay_recv_sems.at[0], 4).wait_send()
                slice_rdma(vstage, 5, rbufs.at[1], vsend_sems.at[5],
                           relay_recv_sems.at[1], 4).wait_send()
                slice_rdma(vstage, 7, rbufs.at[2], vsend_sems.at[7],
                           relay_recv_sems.at[2], 4).wait_send()
                kdir(4).start()
                vdir(4).start()

                slice_rdma(kstage, 6, rbufs.at[0], ksend_sems.at[6],
                           relay_recv_sems.at[0], 3).wait_send()
                slice_rdma(vstage, 6, rbufs.at[1], vsend_sems.at[6],
                           relay_recv_sems.at[1], 3).wait_send()
                kdir(7).wait_send()
                kdir(3).start()
                vdir(3).start()

        def ph1_relay():
            @pl.when(me == 4)
            def _():
                relay_wait(0)
                fwd(0, kbuf, krecv_sem, 5).start()
                relay_wait(1)
                fwd(1, vbuf, vrecv_sem, 5).start()
                relay_wait(2)
                fwd(2, vbuf, vrecv_sem, 7).start()

            @pl.when(me == 3)
            def _():
                relay_wait(0)
                fwd(0, kbuf, krecv_sem, 6).start()
                relay_wait(1)
                fwd(1, vbuf, vrecv_sem, 6).start()

        def ph2_qproj():
            wq = wq_ref[...].astype(BF16)
            qs = []
            for b in range(B):
                xb = x_ref[b].astype(BF16)
                q = jnp.dot(xb, wq, preferred_element_type=jnp.float32)
                qs.append((q * 0.125).astype(BF16))
            return qs

        def ph_k_wait():
            @pl.when(me != 0)
            def _():
                pltpu.make_async_remote_copy(
                    src_ref=kbuf, dst_ref=kbuf,
                    send_sem=ksend_sems.at[0], recv_sem=krecv_sem,
                    device_id=(0,), device_id_type=MESH).wait_recv()

        def ph_v_wait():
            @pl.when(me != 0)
            def _():
                pltpu.make_async_remote_copy(
                    src_ref=vbuf, dst_ref=vbuf,
                    send_sem=vsend_sems.at[0], recv_sem=vrecv_sem,
                    device_id=(0,), device_id_type=MESH).wait_recv()

        def ph3a_scores(qs):
            row = lax.broadcasted_iota(jnp.int32, (SQ, SKV_LOC), 0) // BLK
            col = lax.broadcasted_iota(jnp.int32, (SQ, SKV_LOC), 1) // BLK
            mask = col <= row
            ws = []
            for b in range(B):
                kb2 = kbuf[b]
                wb = []
                for h in range(HQ_LOC):
                    qh = qs[b][:, DH * h:DH * (h + 1)]
                    kh = kb2[DH * h:DH * (h + 1), :]
                    s = lax.dot_general(
                        qh, kh, (((1,), (0,)), ((), ())),
                        preferred_element_type=jnp.float32)
                    s = jnp.where(mask, s, -1e9)
                    mx = jnp.max(s, axis=1, keepdims=True)
                    w = jnp.exp(s - mx)
                    wb.append(
                        (w / jnp.sum(w, axis=1, keepdims=True)).astype(BF16))
                ws.append(wb)
            return ws

        def ph3b_ctx(ws):
            wo = wo_ref[...].astype(BF16)
            for b in range(B):
                vb2 = vbuf[b]
                acc = jnp.zeros((SQ, DM), jnp.float32)
                for h in range(HQ_LOC):
                    ch = lax.dot_general(
                        ws[b][h], vb2[DH * h:DH * (h + 1), :],
                        (((1,), (1,)), ((), ())),
                        preferred_element_type=jnp.float32)
                    acc = acc + jnp.dot(
                        ch.astype(BF16), wo[DH * h:DH * (h + 1), :],
                        preferred_element_type=jnp.float32)
                part[b] = acc.astype(BF16)

        def rs_rdma(s, d):
            return pltpu.make_async_remote_copy(
                src_ref=part.at[:, pl.ds(SC * d, SC), :],
                dst_ref=rsbuf.at[s],
                send_sem=rssend_sems.at[d], recv_sem=rsrecv_sems.at[s],
                device_id=(d,), device_id_type=MESH)

        def ag_rdma(s, d):
            return pltpu.make_async_remote_copy(
                src_ref=agstage, dst_ref=agbuf.at[s],
                send_sem=agsend_sems.at[d], recv_sem=agrecv_sems.at[s],
                device_id=(d,), device_id_type=MESH)

        def ph4_rs_issue():
            for s in range(N_DEV):
                @pl.when(me == s)
                def _(s=s):
                    rsbuf[s] = part[:, SC * s:SC * (s + 1), :]
                    for d in range(N_DEV):
                        if d != s:
                            rs_rdma(s, d).start()

        def ph4_rs_wait():
            for s in range(N_DEV):
                @pl.when(me != s)
                def _(s=s):
                    rs_rdma(s, 0).wait_recv()

        def ph4_reduce():
            red = rsbuf[0].astype(jnp.float32)
            for s in range(1, N_DEV):
                red = red + rsbuf[s].astype(jnp.float32)
            agstage[...] = red.astype(BF16)
            return red

        def ph4_ag_issue(red):
            for s in range(N_DEV):
                @pl.when(me == s)
                def _(s=s):
                    out_ref[:, SC * s:SC * (s + 1), :] = red
                    for d in range(N_DEV):
                        if d != s:
                            ag_rdma(s, d).start()
            for s in range(N_DEV):
                @pl.when(me == s)
                def _(s=s):
                    for d in range(N_DEV):
                        if d != s:
                            rs_rdma(s, d).wait_send()

        def ph4_ag_wait():
            for s in range(N_DEV):
                @pl.when(me != s)
                def _(s=s):
                    ag_rdma(s, 0).wait_recv()
                    out_ref[:, SC * s:SC * (s + 1), :] = (
                        agbuf[s].astype(jnp.float32))
            for s in range(N_DEV):
                @pl.when(me == s)
                def _(s=s):
                    for d in range(N_DEV):
                        if d != s:
                            ag_rdma(s, d).wait_send()

        def ph5_drain():
            @pl.when(me == 0)
            def _():
                for d in (1, 2, 3, 4):
                    vdir(d).wait_send()
                kdir(3).wait_send()
                kdir(4).wait_send()

            @pl.when(me == 4)
            def _():
                fwd(0, kbuf, krecv_sem, 5).wait_send()
                fwd(1, vbuf, vrecv_sem, 5).wait_send()
                fwd(2, vbuf, vrecv_sem, 7).wait_send()

            @pl.when(me == 3)
            def _():
                fwd(0, kbuf, krecv_sem, 6).wait_send()
                fwd(1, vbuf, vrecv_sem, 6).wait_send()

        with jax.named_scope("p1_bcast_issue"):
            ph1_bcast_issue()
        with jax.named_scope("p1_relay"):
            ph1_relay()
        with jax.named_scope("p2_qproj"):
            qs = ph2_qproj()
        with jax.named_scope("p2_k_wait"):
            ph_k_wait()
        with jax.named_scope("p3_scores"):
            ws = ph3a_scores(qs)
        with jax.named_scope("p2_v_wait"):
            ph_v_wait()
        with jax.named_scope("p3_ctx"):
            ph3b_ctx(ws)
        with jax.named_scope("p4_rs_issue"):
            ph4_rs_issue()
        with jax.named_scope("p4_rs_wait"):
            ph4_rs_wait()
        with jax.named_scope("p4_reduce"):
            red = ph4_reduce()
        with jax.named_scope("p4_ag_issue"):
            ph4_ag_issue(red)
        with jax.named_scope("p4_ag_wait"):
            ph4_ag_wait()
        with jax.named_scope("p5_drain"):
            ph5_drain()

    return pl.pallas_call(
        body,
        out_shape=jax.ShapeDtypeStruct((B, SQ, DM), jnp.float32),
        in_specs=[
            pl.BlockSpec(memory_space=pltpu.VMEM),
            pl.BlockSpec(memory_space=pltpu.VMEM),
            pl.BlockSpec(memory_space=pl.ANY),
            pl.BlockSpec(memory_space=pl.ANY),
            pl.BlockSpec(memory_space=pltpu.VMEM),
        ],
        out_specs=pl.BlockSpec(memory_space=pltpu.VMEM),
        scratch_shapes=[
            pltpu.VMEM((B, HQ * DH, SKV_LOC), jnp.float32),
            pltpu.VMEM((B, HQ * DH, SKV_LOC), jnp.float32),
            pltpu.VMEM((B, HQ * DH, SKV_LOC), BF16),
            pltpu.VMEM((B, HQ * DH, SKV_LOC), BF16),
            pltpu.VMEM((B, HQ_LOC * DH, SKV_LOC), BF16),
            pltpu.VMEM((B, HQ_LOC * DH, SKV_LOC), BF16),
            pltpu.VMEM((3, B, HQ_LOC * DH, SKV_LOC), BF16),
            pltpu.VMEM((B, SQ, DM), BF16),
            pltpu.VMEM((N_DEV, B, SC, DM), BF16),
            pltpu.VMEM((B, SC, DM), BF16),
            pltpu.VMEM((N_DEV, B, SC, DM), BF16),
            pltpu.SemaphoreType.DMA((2,)),
            pltpu.SemaphoreType.DMA((N_DEV,)),
            pltpu.SemaphoreType.DMA((N_DEV,)),
            pltpu.SemaphoreType.DMA,
            pltpu.SemaphoreType.DMA,
            pltpu.SemaphoreType.DMA((3,)),
            pltpu.SemaphoreType.DMA((3,)),
            pltpu.SemaphoreType.DMA((N_DEV,)),
            pltpu.SemaphoreType.DMA((N_DEV,)),
            pltpu.SemaphoreType.DMA((N_DEV,)),
            pltpu.SemaphoreType.DMA((N_DEV,)),
        ],
    )(x, Wq,
       jnp.transpose(K_ext, (0, 2, 3, 1)).reshape(B, HQ * DH, SKV_LOC),
       jnp.transpose(V_ext, (0, 2, 3, 1)).reshape(B, HQ * DH, SKV_LOC),
       Wo)


# device time: 42780 ns/iter; 1.0043x vs baseline; 1.0043x over previous
import jax
import jax.numpy as jnp
from jax import lax
from jax.experimental import pallas as pl
from jax.experimental.pallas import tpu as pltpu

N_DEV = 8
B = 2
SQ = 256
DM = 512
DH = 64
HQ_LOC = 4
HQ = N_DEV * HQ_LOC
SKV_LOC = 256
BLK = 64
BF16 = jnp.bfloat16
SC = SQ // N_DEV

MESH = pl.DeviceIdType.MESH


def kernel(x, Wq, K_ext, V_ext, Wo):
    def body(x_ref, wq_ref, k_ref, v_ref, wo_ref, out_ref,
             xv, wqv, wov, kf32, vf32, kstage, vstage, kbuf, vbuf, rbufs,
             part, rsbuf, agstage, agbuf,
             copy_sems, ksend_sems, vsend_sems, krecv_sem, vrecv_sem,
             relay_recv_sems, fwd_send_sems,
             rssend_sems, rsrecv_sems, agsend_sems, agrecv_sems):
        me = lax.axis_index("i")

        def fetch(ref, dst, slot):
            return pltpu.make_async_copy(ref, dst, copy_sems.at[slot])

        def slice_rdma(stage, d, dst, send_sem, recv_sem, target):
            return pltpu.make_async_remote_copy(
                src_ref=stage.at[:, pl.ds(HQ_LOC * DH * d, HQ_LOC * DH), :],
                dst_ref=dst, send_sem=send_sem, recv_sem=recv_sem,
                device_id=(target,), device_id_type=MESH)

        def kdir(d):
            return slice_rdma(kstage, d, kbuf, ksend_sems.at[d], krecv_sem, d)

        def vdir(d):
            return slice_rdma(vstage, d, vbuf, vsend_sems.at[d], vrecv_sem, d)

        def fwd(slot, dst, recv_sem, target):
            return pltpu.make_async_remote_copy(
                src_ref=rbufs.at[slot], dst_ref=dst,
                send_sem=fwd_send_sems.at[slot], recv_sem=recv_sem,
                device_id=(target,), device_id_type=MESH)

        def relay_wait(slot):
            pltpu.make_async_remote_copy(
                src_ref=rbufs.at[slot], dst_ref=rbufs.at[slot],
                send_sem=fwd_send_sems.at[slot],
                recv_sem=relay_recv_sems.at[slot],
                device_id=(0,), device_id_type=MESH).wait_recv()

        def ph0_fetch():
            fetch(x_ref, xv, 2).start()
            fetch(wq_ref, wqv, 3).start()
            fetch(wo_ref, wov, 4).start()

        def ph1_bcast_issue():
            @pl.when(me == 0)
            def _():
                ck = fetch(k_ref, kf32, 0)
                cv = fetch(v_ref, vf32, 1)
                ck.start()
                cv.start()
                ck.wait()
                cv.wait()
                kstage[...] = kf32[...].astype(BF16)
                vstage[...] = vf32[...].astype(BF16)

                slice_rdma(kstage, 5, rbufs.at[0], ksend_sems.at[5],
                           relay_recv_sems.at[0], 4).start()
                slice_rdma(vstage, 5, rbufs.at[1], vsend_sems.at[5],
                           relay_recv_sems.at[1], 4).start()
                slice_rdma(vstage, 7, rbufs.at[2], vsend_sems.at[7],
                           relay_recv_sems.at[2], 4).start()
                slice_rdma(kstage, 6, rbufs.at[0], ksend_sems.at[6],
                           relay_recv_sems.at[0], 3).start()
                slice_rdma(vstage, 6, rbufs.at[1], vsend_sems.at[6],
                           relay_recv_sems.at[1], 3).start()
                kdir(7).start()
                kdir(2).start()
                kdir(1).start()

                kbuf[...] = kstage[:, 0:HQ_LOC * DH, :]
                vbuf[...] = vstage[:, 0:HQ_LOC * DH, :]

                kdir(2).wait_send()
                kdir(1).wait_send()
                vdir(2).start()
                vdir(1).start()

                slice_rdma(kstage, 5, rbufs.at[0], ksend_sems.at[5],
                           relay_recv_sems.at[0], 4).wait_send()
                slice_rdma(vstage, 5, rbufs.at[1], vsend_sems.at[5],
                           relay_recv_sems.at[1], 4).wait_send()
                slice_rdma(vstage, 7, rbufs.at[2], vsend_sems.at[7],
                           relay_recv_sems.at[2], 4).wait_send()
                kdir(4).start()
                vdir(4).start()

                slice_rdma(kstage, 6, rbufs.at[0], ksend_sems.at[6],
                           relay_recv_sems.at[0], 3).wait_send()
                slice_rdma(vstage, 6, rbufs.at[1], vsend_sems.at[6],
                           relay_recv_sems.at[1], 3).wait_send()
                kdir(7).wait_send()
                kdir(3).start()
                vdir(3).start()

        def ph1_relay():
            @pl.when(me == 4)
            def _():
                relay_wait(0)
                fwd(0, kbuf, krecv_sem, 5).start()
                relay_wait(1)
                fwd(1, vbuf, vrecv_sem, 5).start()
                relay_wait(2)
                fwd(2, vbuf, vrecv_sem, 7).start()

            @pl.when(me == 3)
            def _():
                relay_wait(0)
                fwd(0, kbuf, krecv_sem, 6).start()
                relay_wait(1)
                fwd(1, vbuf, vrecv_sem, 6).start()

        def ph2_qproj():
            fetch(x_ref, xv, 2).wait()
            fetch(wq_ref, wqv, 3).wait()
            wq = wqv[...].astype(BF16)
            qs = []
            for b in range(B):
                xb = xv[b].astype(BF16)
                q = jnp.dot(xb, wq, preferred_element_type=jnp.float32)
                qs.append((q * 0.125).astype(BF16))
            return qs

        def ph_k_wait():
            @pl.when(me != 0)
            def _():
                pltpu.make_async_remote_copy(
                    src_ref=kbuf, dst_ref=kbuf,
                    send_sem=ksend_sems.at[0], recv_sem=krecv_sem,
                    device_id=(0,), device_id_type=MESH).wait_recv()

        def ph_v_wait():
            @pl.when(me != 0)
            def _():
                pltpu.make_async_remote_copy(
                    src_ref=vbuf, dst_ref=vbuf,
                    send_sem=vsend_sems.at[0], recv_sem=vrecv_sem,
                    device_id=(0,), device_id_type=MESH).wait_recv()

        def ph3a_scores(qs):
            row = lax.broadcasted_iota(jnp.int32, (SQ, SKV_LOC), 0) // BLK
            col = lax.broadcasted_iota(jnp.int32, (SQ, SKV_LOC), 1) // BLK
            mask = col <= row
            ws = []
            for b in range(B):
                kb2 = kbuf[b]
                wb = []
                for h in range(HQ_LOC):
                    qh = qs[b][:, DH * h:DH * (h + 1)]
                    kh = kb2[DH * h:DH * (h + 1), :]
                    s = lax.dot_general(
                        qh, kh, (((1,), (0,)), ((), ())),
                        preferred_element_type=jnp.float32)
                    s = jnp.where(mask, s, -1e9)
                    mx = jnp.max(s, axis=1, keepdims=True)
                    w = jnp.exp(s - mx)
                    wb.append(
                        (w / jnp.sum(w, axis=1, keepdims=True)).astype(BF16))
                ws.append(wb)
            return ws

        def ph3b_ctx(ws):
            fetch(wo_ref, wov, 4).wait()
            wo = wov[...].astype(BF16)
            for b in range(B):
                vb2 = vbuf[b]
                acc = jnp.zeros((SQ, DM), jnp.float32)
                for h in range(HQ_LOC):
                    ch = lax.dot_general(
                        ws[b][h], vb2[DH * h:DH * (h + 1), :],
                        (((1,), (1,)), ((), ())),
                        preferred_element_type=jnp.float32)
                    acc = acc + jnp.dot(
                        ch.astype(BF16), wo[DH * h:DH * (h + 1), :],
                        preferred_element_type=jnp.float32)
                part[b] = acc.astype(BF16)

        def rs_rdma(s, d):
            return pltpu.make_async_remote_copy(
                src_ref=part.at[:, pl.ds(SC * d, SC), :],
                dst_ref=rsbuf.at[s],
                send_sem=rssend_sems.at[d], recv_sem=rsrecv_sems.at[s],
                device_id=(d,), device_id_type=MESH)

        def ag_rdma(s, d):
            return pltpu.make_async_remote_copy(
                src_ref=agstage, dst_ref=agbuf.at[s],
                send_sem=agsend_sems.at[d], recv_sem=agrecv_sems.at[s],
                device_id=(d,), device_id_type=MESH)

        def ph4_rs_issue():
            for s in range(N_DEV):
                @pl.when(me == s)
                def _(s=s):
                    rsbuf[s] = part[:, SC * s:SC * (s + 1), :]
                    for d in range(N_DEV):
                        if d != s:
                            rs_rdma(s, d).start()

        def ph4_rs_wait():
            for s in range(N_DEV):
                @pl.when(me != s)
                def _(s=s):
                    rs_rdma(s, 0).wait_recv()

        def ph4_reduce():
            red = rsbuf[0].astype(jnp.float32)
            for s in range(1, N_DEV):
                red = red + rsbuf[s].astype(jnp.float32)
            agstage[...] = red.astype(BF16)
            return red

        def ph4_ag_issue(red):
            for s in range(N_DEV):
                @pl.when(me == s)
                def _(s=s):
                    out_ref[:, SC * s:SC * (s + 1), :] = red
                    for d in range(N_DEV):
                        if d != s:
                            ag_rdma(s, d).start()
            for s in range(N_DEV):
                @pl.when(me == s)
                def _(s=s):
                    for d in range(N_DEV):
                        if d != s:
                            rs_rdma(s, d).wait_send()

        def ph4_ag_wait():
            for s in range(N_DEV):
                @pl.when(me != s)
                def _(s=s):
                    ag_rdma(s, 0).wait_recv()
                    out_ref[:, SC * s:SC * (s + 1), :] = (
                        agbuf[s].astype(jnp.float32))
            for s in range(N_DEV):
                @pl.when(me == s)
                def _(s=s):
                    for d in range(N_DEV):
                        if d != s:
                            ag_rdma(s, d).wait_send()

        def ph5_drain():
            @pl.when(me == 0)
            def _():
                for d in (1, 2, 3, 4):
                    vdir(d).wait_send()
                kdir(3).wait_send()
                kdir(4).wait_send()

            @pl.when(me == 4)
            def _():
                fwd(0, kbuf, krecv_sem, 5).wait_send()
                fwd(1, vbuf, vrecv_sem, 5).wait_send()
                fwd(2, vbuf, vrecv_sem, 7).wait_send()

            @pl.when(me == 3)
            def _():
                fwd(0, kbuf, krecv_sem, 6).wait_send()
                fwd(1, vbuf, vrecv_sem, 6).wait_send()

        with jax.named_scope("p0_fetch"):
            ph0_fetch()
        with jax.named_scope("p1_bcast_issue"):
            ph1_bcast_issue()
        with jax.named_scope("p1_relay"):
            ph1_relay()
        with jax.named_scope("p2_qproj"):
            qs = ph2_qproj()
        with jax.named_scope("p2_k_wait"):
            ph_k_wait()
        with jax.named_scope("p3_scores"):
            ws = ph3a_scores(qs)
        with jax.named_scope("p2_v_wait"):
            ph_v_wait()
        with jax.named_scope("p3_ctx"):
            ph3b_ctx(ws)
        with jax.named_scope("p4_rs_issue"):
            ph4_rs_issue()
        with jax.named_scope("p4_rs_wait"):
            ph4_rs_wait()
        with jax.named_scope("p4_reduce"):
            red = ph4_reduce()
        with jax.named_scope("p4_ag_issue"):
            ph4_ag_issue(red)
        with jax.named_scope("p4_ag_wait"):
            ph4_ag_wait()
        with jax.named_scope("p5_drain"):
            ph5_drain()

    return pl.pallas_call(
        body,
        out_shape=jax.ShapeDtypeStruct((B, SQ, DM), jnp.float32),
        in_specs=[pl.BlockSpec(memory_space=pl.ANY)] * 5,
        out_specs=pl.BlockSpec(memory_space=pltpu.VMEM),
        scratch_shapes=[
            pltpu.VMEM((B, SQ, DM), jnp.float32),
            pltpu.VMEM((DM, HQ_LOC * DH), jnp.float32),
            pltpu.VMEM((HQ_LOC * DH, DM), jnp.float32),
            pltpu.VMEM((B, HQ * DH, SKV_LOC), jnp.float32),
            pltpu.VMEM((B, HQ * DH, SKV_LOC), jnp.float32),
            pltpu.VMEM((B, HQ * DH, SKV_LOC), BF16),
            pltpu.VMEM((B, HQ * DH, SKV_LOC), BF16),
            pltpu.VMEM((B, HQ_LOC * DH, SKV_LOC), BF16),
            pltpu.VMEM((B, HQ_LOC * DH, SKV_LOC), BF16),
            pltpu.VMEM((3, B, HQ_LOC * DH, SKV_LOC), BF16),
            pltpu.VMEM((B, SQ, DM), BF16),
            pltpu.VMEM((N_DEV, B, SC, DM), BF16),
            pltpu.VMEM((B, SC, DM), BF16),
            pltpu.VMEM((N_DEV, B, SC, DM), BF16),
            pltpu.SemaphoreType.DMA((5,)),
            pltpu.SemaphoreType.DMA((N_DEV,)),
            pltpu.SemaphoreType.DMA((N_DEV,)),
            pltpu.SemaphoreType.DMA,
            pltpu.SemaphoreType.DMA,
            pltpu.SemaphoreType.DMA((3,)),
            pltpu.SemaphoreType.DMA((3,)),
            pltpu.SemaphoreType.DMA((N_DEV,)),
            pltpu.SemaphoreType.DMA((N_DEV,)),
            pltpu.SemaphoreType.DMA((N_DEV,)),
            pltpu.SemaphoreType.DMA((N_DEV,)),
        ],
    )(x, Wq,
       jnp.transpose(K_ext, (0, 2, 3, 1)).reshape(B, HQ * DH, SKV_LOC),
       jnp.transpose(V_ext, (0, 2, 3, 1)).reshape(B, HQ * DH, SKV_LOC),
       Wo)


# device time: 42380 ns/iter; 1.0138x vs baseline; 1.0094x over previous
import jax
import jax.numpy as jnp
from jax import lax
from jax.experimental import pallas as pl
from jax.experimental.pallas import tpu as pltpu

N_DEV = 8
B = 2
SQ = 256
DM = 512
DH = 64
HQ_LOC = 4
HQ = N_DEV * HQ_LOC
SKV_LOC = 256
BLK = 64
BF16 = jnp.bfloat16
SC = SQ // N_DEV

MESH = pl.DeviceIdType.MESH


def kernel(x, Wq, K_ext, V_ext, Wo):
    def body(x_ref, wq_ref, k_ref, v_ref, wo_ref, out_ref,
             kstage, vstage, kbuf, vbuf, rbufs,
             part, rsbuf, agstage, agbuf,
             ksend_sems, vsend_sems, krecv_sem, vrecv_sem,
             relay_recv_sems, fwd_send_sems,
             rssend_sems, rsrecv_sems, agsend_sems, agrecv_sems):
        me = lax.axis_index("i")

        def slice_rdma(stage, d, dst, send_sem, recv_sem, target):
            return pltpu.make_async_remote_copy(
                src_ref=stage.at[:, pl.ds(HQ_LOC * DH * d, HQ_LOC * DH), :],
                dst_ref=dst, send_sem=send_sem, recv_sem=recv_sem,
                device_id=(target,), device_id_type=MESH)

        def kdir(d):
            return slice_rdma(kstage, d, kbuf, ksend_sems.at[d], krecv_sem, d)

        def vdir(d):
            return slice_rdma(vstage, d, vbuf, vsend_sems.at[d], vrecv_sem, d)

        def fwd(slot, dst, recv_sem, target):
            return pltpu.make_async_remote_copy(
                src_ref=rbufs.at[slot], dst_ref=dst,
                send_sem=fwd_send_sems.at[slot], recv_sem=recv_sem,
                device_id=(target,), device_id_type=MESH)

        def relay_wait(slot):
            pltpu.make_async_remote_copy(
                src_ref=rbufs.at[slot], dst_ref=rbufs.at[slot],
                send_sem=fwd_send_sems.at[slot],
                recv_sem=relay_recv_sems.at[slot],
                device_id=(0,), device_id_type=MESH).wait_recv()

        def ph1_bcast_issue():
            @pl.when(me == 0)
            def _():
                kstage[...] = k_ref[...].astype(BF16)
                vstage[...] = v_ref[...].astype(BF16)

                slice_rdma(kstage, 5, rbufs.at[0], ksend_sems.at[5],
                           relay_recv_sems.at[0], 4).start()
                slice_rdma(vstage, 5, rbufs.at[1], vsend_sems.at[5],
                           relay_recv_sems.at[1], 4).start()
                slice_rdma(vstage, 7, rbufs.at[2], vsend_sems.at[7],
                           relay_recv_sems.at[2], 4).start()
                slice_rdma(kstage, 6, rbufs.at[0], ksend_sems.at[6],
                           relay_recv_sems.at[0], 3).start()
                slice_rdma(vstage, 6, rbufs.at[1], vsend_sems.at[6],
                           relay_recv_sems.at[1], 3).start()
                kdir(7).start()
                kdir(2).start()
                kdir(1).start()

                kbuf[...] = kstage[:, 0:HQ_LOC * DH, :]
                vbuf[...] = vstage[:, 0:HQ_LOC * DH, :]

                kdir(2).wait_send()
                kdir(1).wait_send()
                vdir(2).start()
                vdir(1).start()

                slice_rdma(kstage, 5, rbufs.at[0], ksend_sems.at[5],
                           relay_recv_sems.at[0], 4).wait_send()
                slice_rdma(vstage, 5, rbufs.at[1], vsend_sems.at[5],
                           relay_recv_sems.at[1], 4).wait_send()
                slice_rdma(vstage, 7, rbufs.at[2], vsend_sems.at[7],
                           relay_recv_sems.at[2], 4).wait_send()
                kdir(4).start()
                vdir(4).start()

                slice_rdma(kstage, 6, rbufs.at[0], ksend_sems.at[6],
                           relay_recv_sems.at[0], 3).wait_send()
                slice_rdma(vstage, 6, rbufs.at[1], vsend_sems.at[6],
                           relay_recv_sems.at[1], 3).wait_send()
                kdir(7).wait_send()
                kdir(3).start()
                vdir(3).start()

        def ph1_relay():
            @pl.when(me == 4)
            def _():
                relay_wait(0)
                fwd(0, kbuf, krecv_sem, 5).start()
                relay_wait(1)
                fwd(1, vbuf, vrecv_sem, 5).start()
                relay_wait(2)
                fwd(2, vbuf, vrecv_sem, 7).start()

            @pl.when(me == 3)
            def _():
                relay_wait(0)
                fwd(0, kbuf, krecv_sem, 6).start()
                relay_wait(1)
                fwd(1, vbuf, vrecv_sem, 6).start()

        def ph2_qproj():
            wq = wq_ref[...].astype(BF16)
            qs = []
            for b in range(B):
                xb = x_ref[b].astype(BF16)
                q = jnp.dot(xb, wq, preferred_element_type=jnp.float32)
                qs.append((q * 0.125).astype(BF16))
            return qs

        def ph_k_wait():
            @pl.when(me != 0)
            def _():
                pltpu.make_async_remote_copy(
                    src_ref=kbuf, dst_ref=kbuf,
                    send_sem=ksend_sems.at[0], recv_sem=krecv_sem,
                    device_id=(0,), device_id_type=MESH).wait_recv()

        def ph_v_wait():
            @pl.when(me != 0)
            def _():
                pltpu.make_async_remote_copy(
                    src_ref=vbuf, dst_ref=vbuf,
                    send_sem=vsend_sems.at[0], recv_sem=vrecv_sem,
                    device_id=(0,), device_id_type=MESH).wait_recv()

        def ph3a_scores(qs):
            row = lax.broadcasted_iota(jnp.int32, (SQ, SKV_LOC), 0) // BLK
            col = lax.broadcasted_iota(jnp.int32, (SQ, SKV_LOC), 1) // BLK
            mask = col <= row
            ws = []
            for b in range(B):
                kb2 = kbuf[b]
                wb = []
                for h in range(HQ_LOC):
                    qh = qs[b][:, DH * h:DH * (h + 1)]
                    kh = kb2[DH * h:DH * (h + 1), :]
                    s = lax.dot_general(
                        qh, kh, (((1,), (0,)), ((), ())),
                        preferred_element_type=jnp.float32)
                    s = jnp.where(mask, s, -1e9)
                    mx = jnp.max(s, axis=1, keepdims=True)
                    w = jnp.exp(s - mx)
                    wb.append(
                        (w / jnp.sum(w, axis=1, keepdims=True)).astype(BF16))
                ws.append(wb)
            return ws

        def ph3b_ctx(ws):
            wo = wo_ref[...].astype(BF16)
            for b in range(B):
                vb2 = vbuf[b]
                acc = jnp.zeros((SQ, DM), jnp.float32)
                for h in range(HQ_LOC):
                    ch = lax.dot_general(
                        ws[b][h], vb2[DH * h:DH * (h + 1), :],
                        (((1,), (1,)), ((), ())),
                        preferred_element_type=jnp.float32)
                    acc = acc + jnp.dot(
                        ch.astype(BF16), wo[DH * h:DH * (h + 1), :],
                        preferred_element_type=jnp.float32)
                part[b] = acc.astype(BF16)

        def rs_rdma(s, d):
            return pltpu.make_async_remote_copy(
                src_ref=part.at[:, pl.ds(SC * d, SC), :],
                dst_ref=rsbuf.at[s],
                send_sem=rssend_sems.at[d], recv_sem=rsrecv_sems.at[s],
                device_id=(d,), device_id_type=MESH)

        def ag_rdma(s, d):
            return pltpu.make_async_remote_copy(
                src_ref=agstage, dst_ref=agbuf.at[s],
                send_sem=agsend_sems.at[d], recv_sem=agrecv_sems.at[s],
                device_id=(d,), device_id_type=MESH)

        def ph4_rs_issue():
            for s in range(N_DEV):
                @pl.when(me == s)
                def _(s=s):
                    rsbuf[s] = part[:, SC * s:SC * (s + 1), :]
                    for d in range(N_DEV):
                        if d != s:
                            rs_rdma(s, d).start()

        def ph4_rs_wait():
            for s in range(N_DEV):
                @pl.when(me != s)
                def _(s=s):
                    rs_rdma(s, 0).wait_recv()

        def ph4_reduce():
            red = rsbuf[0].astype(jnp.float32)
            for s in range(1, N_DEV):
                red = red + rsbuf[s].astype(jnp.float32)
            agstage[...] = red.astype(BF16)
            return red

        def ph4_ag_issue(red):
            for s in range(N_DEV):
                @pl.when(me == s)
                def _(s=s):
                    out_ref[:, SC * s:SC * (s + 1), :] = red
                    for d in range(N_DEV):
                        if d != s:
                            ag_rdma(s, d).start()
            for s in range(N_DEV):
                @pl.when(me == s)
                def _(s=s):
                    for d in range(N_DEV):
                        if d != s:
                            rs_rdma(s, d).wait_send()

        def ph4_ag_wait():
            for s in range(N_DEV):
                @pl.when(me != s)
                def _(s=s):
                    ag_rdma(s, 0).wait_recv()
                    out_ref[:, SC * s:SC * (s + 1), :] = (
                        agbuf[s].astype(jnp.float32))
            for s in range(N_DEV):
                @pl.when(me == s)
                def _(s=s):
                    for d in range(N_DEV):
                        if d != s:
                            ag_rdma(s, d).wait_send()

        def ph5_drain():
            @pl.when(me == 0)
            def _():
                for d in (1, 2, 3, 4):
                    vdir(d).wait_send()
                kdir(3).wait_send()
                kdir(4).wait_send()

            @pl.when(me == 4)
            def _():
                fwd(0, kbuf, krecv_sem, 5).wait_send()
                fwd(1, vbuf, vrecv_sem, 5).wait_send()
                fwd(2, vbuf, vrecv_sem, 7).wait_send()

            @pl.when(me == 3)
            def _():
                fwd(0, kbuf, krecv_sem, 6).wait_send()
                fwd(1, vbuf, vrecv_sem, 6).wait_send()

        with jax.named_scope("p1_bcast_issue"):
            ph1_bcast_issue()
        with jax.named_scope("p1_relay"):
            ph1_relay()
        with jax.named_scope("p2_qproj"):
            qs = ph2_qproj()
        with jax.named_scope("p2_k_wait"):
            ph_k_wait()
        with jax.named_scope("p3_scores"):
            ws = ph3a_scores(qs)
        with jax.named_scope("p2_v_wait"):
            ph_v_wait()
        with jax.named_scope("p3_ctx"):
            ph3b_ctx(ws)
        with jax.named_scope("p4_rs_issue"):
            ph4_rs_issue()
        with jax.named_scope("p4_rs_wait"):
            ph4_rs_wait()
        with jax.named_scope("p4_reduce"):
            red = ph4_reduce()
        with jax.named_scope("p4_ag_issue"):
            ph4_ag_issue(red)
        with jax.named_scope("p4_ag_wait"):
            ph4_ag_wait()
        with jax.named_scope("p5_drain"):
            ph5_drain()

    return pl.pallas_call(
        body,
        out_shape=jax.ShapeDtypeStruct((B, SQ, DM), jnp.float32),
        in_specs=[pl.BlockSpec(memory_space=pltpu.VMEM)] * 5,
        out_specs=pl.BlockSpec(memory_space=pltpu.VMEM),
        scratch_shapes=[
            pltpu.VMEM((B, HQ * DH, SKV_LOC), BF16),
            pltpu.VMEM((B, HQ * DH, SKV_LOC), BF16),
            pltpu.VMEM((B, HQ_LOC * DH, SKV_LOC), BF16),
            pltpu.VMEM((B, HQ_LOC * DH, SKV_LOC), BF16),
            pltpu.VMEM((3, B, HQ_LOC * DH, SKV_LOC), BF16),
            pltpu.VMEM((B, SQ, DM), BF16),
            pltpu.VMEM((N_DEV, B, SC, DM), BF16),
            pltpu.VMEM((B, SC, DM), BF16),
            pltpu.VMEM((N_DEV, B, SC, DM), BF16),
            pltpu.SemaphoreType.DMA((N_DEV,)),
            pltpu.SemaphoreType.DMA((N_DEV,)),
            pltpu.SemaphoreType.DMA,
            pltpu.SemaphoreType.DMA,
            pltpu.SemaphoreType.DMA((3,)),
            pltpu.SemaphoreType.DMA((3,)),
            pltpu.SemaphoreType.DMA((N_DEV,)),
            pltpu.SemaphoreType.DMA((N_DEV,)),
            pltpu.SemaphoreType.DMA((N_DEV,)),
            pltpu.SemaphoreType.DMA((N_DEV,)),
        ],
    )(x, Wq,
       jnp.transpose(K_ext, (0, 2, 3, 1)).reshape(B, HQ * DH, SKV_LOC),
       jnp.transpose(V_ext, (0, 2, 3, 1)).reshape(B, HQ * DH, SKV_LOC),
       Wo)


# device time: 41147 ns/iter; 1.0441x vs baseline; 1.0300x over previous
import jax
import jax.numpy as jnp
from jax import lax
from jax.experimental import pallas as pl
from jax.experimental.pallas import tpu as pltpu

N_DEV = 8
B = 2
SQ = 256
DM = 512
DH = 64
HQ_LOC = 4
HQ = N_DEV * HQ_LOC
SKV_LOC = 256
BLK = 64
BF16 = jnp.bfloat16
SC = SQ // N_DEV

MESH = pl.DeviceIdType.MESH


def kernel(x, Wq, K_ext, V_ext, Wo):
    def body(x_ref, wq_ref, k_ref, v_ref, wo_ref, out_ref,
             kf32, vf32, kstage, vstage, kbuf, vbuf, rbufs,
             part, rsbuf, agstage, agbuf,
             copy_sems, ksend_sems, vsend_sems, krecv_sem, vrecv_sem,
             relay_recv_sems, fwd_send_sems,
             rssend_sems, rsrecv_sems, agsend_sems, agrecv_sems):
        me = lax.axis_index("i")

        def slice_rdma(stage, d, dst, send_sem, recv_sem, target):
            return pltpu.make_async_remote_copy(
                src_ref=stage.at[:, pl.ds(HQ_LOC * DH * d, HQ_LOC * DH), :],
                dst_ref=dst, send_sem=send_sem, recv_sem=recv_sem,
                device_id=(target,), device_id_type=MESH)

        def kdir(d):
            return slice_rdma(kstage, d, kbuf, ksend_sems.at[d], krecv_sem, d)

        def vdir(d):
            return slice_rdma(vstage, d, vbuf, vsend_sems.at[d], vrecv_sem, d)

        def fwd(slot, dst, recv_sem, target):
            return pltpu.make_async_remote_copy(
                src_ref=rbufs.at[slot], dst_ref=dst,
                send_sem=fwd_send_sems.at[slot], recv_sem=recv_sem,
                device_id=(target,), device_id_type=MESH)

        def relay_wait(slot):
            pltpu.make_async_remote_copy(
                src_ref=rbufs.at[slot], dst_ref=rbufs.at[slot],
                send_sem=fwd_send_sems.at[slot],
                recv_sem=relay_recv_sems.at[slot],
                device_id=(0,), device_id_type=MESH).wait_recv()

        def ph1_bcast_issue():
            @pl.when(me == 0)
            def _():
                ck = pltpu.make_async_copy(k_ref, kf32, copy_sems.at[0])
                cv = pltpu.make_async_copy(v_ref, vf32, copy_sems.at[1])
                ck.start()
                cv.start()
                ck.wait()
                cv.wait()
                kstage[...] = kf32[...].astype(BF16)
                vstage[...] = vf32[...].astype(BF16)

                slice_rdma(kstage, 5, rbufs.at[0], ksend_sems.at[5],
                           relay_recv_sems.at[0], 4).start()
                slice_rdma(vstage, 5, rbufs.at[1], vsend_sems.at[5],
                           relay_recv_sems.at[1], 4).start()
                slice_rdma(vstage, 7, rbufs.at[2], vsend_sems.at[7],
                           relay_recv_sems.at[2], 4).start()
                slice_rdma(kstage, 6, rbufs.at[0], ksend_sems.at[6],
                           relay_recv_sems.at[0], 3).start()
                slice_rdma(vstage, 6, rbufs.at[1], vsend_sems.at[6],
                           relay_recv_sems.at[1], 3).start()
                kdir(7).start()
                kdir(2).start()
                kdir(1).start()

                kbuf[...] = kstage[:, 0:HQ_LOC * DH, :]
                vbuf[...] = vstage[:, 0:HQ_LOC * DH, :]

                kdir(2).wait_send()
                kdir(1).wait_send()
                vdir(2).start()
                vdir(1).start()

                slice_rdma(kstage, 5, rbufs.at[0], ksend_sems.at[5],
                           relay_recv_sems.at[0], 4).wait_send()
                slice_rdma(vstage, 5, rbufs.at[1], vsend_sems.at[5],
                           relay_recv_sems.at[1], 4).wait_send()
                slice_rdma(vstage, 7, rbufs.at[2], vsend_sems.at[7],
                           relay_recv_sems.at[2], 4).wait_send()
                kdir(4).start()
                vdir(4).start()

                slice_rdma(kstage, 6, rbufs.at[0], ksend_sems.at[6],
                           relay_recv_sems.at[0], 3).wait_send()
                slice_rdma(vstage, 6, rbufs.at[1], vsend_sems.at[6],
                           relay_recv_sems.at[1], 3).wait_send()
                kdir(7).wait_send()
                kdir(3).start()
                vdir(3).start()

        def ph1_relay():
            @pl.when(me == 4)
            def _():
                relay_wait(0)
                fwd(0, kbuf, krecv_sem, 5).start()
                relay_wait(1)
                fwd(1, vbuf, vrecv_sem, 5).start()
                relay_wait(2)
                fwd(2, vbuf, vrecv_sem, 7).start()

            @pl.when(me == 3)
            def _():
                relay_wait(0)
                fwd(0, kbuf, krecv_sem, 6).start()
                relay_wait(1)
                fwd(1, vbuf, vrecv_sem, 6).start()

        def ph2_qproj():
            wq = wq_ref[...].astype(BF16)
            qs = []
            for b in range(B):
                xb = x_ref[b].astype(BF16)
                q = jnp.dot(xb, wq, preferred_element_type=jnp.float32)
                qs.append((q * 0.125).astype(BF16))
            return qs

        def ph_k_wait():
            @pl.when(me != 0)
            def _():
                pltpu.make_async_remote_copy(
                    src_ref=kbuf, dst_ref=kbuf,
                    send_sem=ksend_sems.at[0], recv_sem=krecv_sem,
                    device_id=(0,), device_id_type=MESH).wait_recv()

        def ph_v_wait():
            @pl.when(me != 0)
            def _():
                pltpu.make_async_remote_copy(
                    src_ref=vbuf, dst_ref=vbuf,
                    send_sem=vsend_sems.at[0], recv_sem=vrecv_sem,
                    device_id=(0,), device_id_type=MESH).wait_recv()

        def ph3a_scores(qs):
            row = lax.broadcasted_iota(jnp.int32, (SQ, SKV_LOC), 0) // BLK
            col = lax.broadcasted_iota(jnp.int32, (SQ, SKV_LOC), 1) // BLK
            mask = col <= row
            ws = []
            for b in range(B):
                kb2 = kbuf[b]
                wb = []
                for h in range(HQ_LOC):
                    qh = qs[b][:, DH * h:DH * (h + 1)]
                    kh = kb2[DH * h:DH * (h + 1), :]
                    s = lax.dot_general(
                        qh, kh, (((1,), (0,)), ((), ())),
                        preferred_element_type=jnp.float32)
                    s = jnp.where(mask, s, -1e9)
                    mx = jnp.max(s, axis=1, keepdims=True)
                    w = jnp.exp(s - mx)
                    wb.append(
                        (w / jnp.sum(w, axis=1, keepdims=True)).astype(BF16))
                ws.append(wb)
            return ws

        def ph3b_ctx(ws):
            wo = wo_ref[...].astype(BF16)
            for b in range(B):
                vb2 = vbuf[b]
                acc = jnp.zeros((SQ, DM), jnp.float32)
                for h in range(HQ_LOC):
                    ch = lax.dot_general(
                        ws[b][h], vb2[DH * h:DH * (h + 1), :],
                        (((1,), (1,)), ((), ())),
                        preferred_element_type=jnp.float32)
                    acc = acc + jnp.dot(
                        ch.astype(BF16), wo[DH * h:DH * (h + 1), :],
                        preferred_element_type=jnp.float32)
                part[b] = acc.astype(BF16)

        def rs_rdma(s, d):
            return pltpu.make_async_remote_copy(
                src_ref=part.at[:, pl.ds(SC * d, SC), :],
                dst_ref=rsbuf.at[s],
                send_sem=rssend_sems.at[d], recv_sem=rsrecv_sems.at[s],
                device_id=(d,), device_id_type=MESH)

        def ag_rdma(s, d):
            return pltpu.make_async_remote_copy(
                src_ref=agstage, dst_ref=agbuf.at[s],
                send_sem=agsend_sems.at[d], recv_sem=agrecv_sems.at[s],
                device_id=(d,), device_id_type=MESH)

        def ph4_rs_issue():
            for s in range(N_DEV):
                @pl.when(me == s)
                def _(s=s):
                    rsbuf[s] = part[:, SC * s:SC * (s + 1), :]
                    for d in range(N_DEV):
                        if d != s:
                            rs_rdma(s, d).start()

        def ph4_rs_wait():
            for s in range(N_DEV):
                @pl.when(me != s)
                def _(s=s):
                    rs_rdma(s, 0).wait_recv()

        def ph4_reduce():
            red = rsbuf[0].astype(jnp.float32)
            for s in range(1, N_DEV):
                red = red + rsbuf[s].astype(jnp.float32)
            agstage[...] = red.astype(BF16)
            return red

        def ph4_ag_issue(red):
            for s in range(N_DEV):
                @pl.when(me == s)
                def _(s=s):
                    out_ref[:, SC * s:SC * (s + 1), :] = red
                    for d in range(N_DEV):
                        if d != s:
                            ag_rdma(s, d).start()
            for s in range(N_DEV):
                @pl.when(me == s)
                def _(s=s):
                    for d in range(N_DEV):
                        if d != s:
                            rs_rdma(s, d).wait_send()

        def ph4_ag_wait():
            for s in range(N_DEV):
                @pl.when(me != s)
                def _(s=s):
                    ag_rdma(s, 0).wait_recv()
                    out_ref[:, SC * s:SC * (s + 1), :] = (
                        agbuf[s].astype(jnp.float32))
            for s in range(N_DEV):
                @pl.when(me == s)
                def _(s=s):
                    for d in range(N_DEV):
                        if d != s:
                            ag_rdma(s, d).wait_send()

        def ph5_drain():
            @pl.when(me == 0)
            def _():
                for d in (1, 2, 3, 4):
                    vdir(d).wait_send()
                kdir(3).wait_send()
                kdir(4).wait_send()

            @pl.when(me == 4)
            def _():
                fwd(0, kbuf, krecv_sem, 5).wait_send()
                fwd(1, vbuf, vrecv_sem, 5).wait_send()
                fwd(2, vbuf, vrecv_sem, 7).wait_send()

            @pl.when(me == 3)
            def _():
                fwd(0, kbuf, krecv_sem, 6).wait_send()
                fwd(1, vbuf, vrecv_sem, 6).wait_send()

        with jax.named_scope("p1_bcast_issue"):
            ph1_bcast_issue()
        with jax.named_scope("p1_relay"):
            ph1_relay()
        with jax.named_scope("p2_qproj"):
            qs = ph2_qproj()
        with jax.named_scope("p2_k_wait"):
            ph_k_wait()
        with jax.named_scope("p3_scores"):
            ws = ph3a_scores(qs)
        with jax.named_scope("p2_v_wait"):
            ph_v_wait()
        with jax.named_scope("p3_ctx"):
            ph3b_ctx(ws)
        with jax.named_scope("p4_rs_issue"):
            ph4_rs_issue()
        with jax.named_scope("p4_rs_wait"):
            ph4_rs_wait()
        with jax.named_scope("p4_reduce"):
            red = ph4_reduce()
        with jax.named_scope("p4_ag_issue"):
            ph4_ag_issue(red)
        with jax.named_scope("p4_ag_wait"):
            ph4_ag_wait()
        with jax.named_scope("p5_drain"):
            ph5_drain()

    return pl.pallas_call(
        body,
        out_shape=jax.ShapeDtypeStruct((B, SQ, DM), jnp.float32),
        in_specs=[
            pl.BlockSpec(memory_space=pltpu.VMEM),
            pl.BlockSpec(memory_space=pltpu.VMEM),
            pl.BlockSpec(memory_space=pl.ANY),
            pl.BlockSpec(memory_space=pl.ANY),
            pl.BlockSpec(memory_space=pltpu.VMEM),
        ],
        out_specs=pl.BlockSpec(memory_space=pltpu.VMEM),
        scratch_shapes=[
            pltpu.VMEM((B, HQ * DH, SKV_LOC), jnp.float32),
            pltpu.VMEM((B, HQ * DH, SKV_LOC), jnp.float32),
            pltpu.VMEM((B, HQ * DH, SKV_LOC), BF16),
            pltpu.VMEM((B, HQ * DH, SKV_LOC), BF16),
            pltpu.VMEM((B, HQ_LOC * DH, SKV_LOC), BF16),
            pltpu.VMEM((B, HQ_LOC * DH, SKV_LOC), BF16),
            pltpu.VMEM((3, B, HQ_LOC * DH, SKV_LOC), BF16),
            pltpu.VMEM((B, SQ, DM), BF16),
            pltpu.VMEM((N_DEV, B, SC, DM), BF16),
            pltpu.VMEM((B, SC, DM), BF16),
            pltpu.VMEM((N_DEV, B, SC, DM), BF16),
            pltpu.SemaphoreType.DMA((2,)),
            pltpu.SemaphoreType.DMA((N_DEV,)),
            pltpu.SemaphoreType.DMA((N_DEV,)),
            pltpu.SemaphoreType.DMA,
            pltpu.SemaphoreType.DMA,
            pltpu.SemaphoreType.DMA((3,)),
            pltpu.SemaphoreType.DMA((3,)),
            pltpu.SemaphoreType.DMA((N_DEV,)),
            pltpu.SemaphoreType.DMA((N_DEV,)),
            pltpu.SemaphoreType.DMA((N_DEV,)),
            pltpu.SemaphoreType.DMA((N_DEV,)),
        ],
    )(x, Wq,
       pltpu.with_memory_space_constraint(
           jnp.transpose(K_ext, (0, 2, 3, 1)).reshape(B, HQ * DH, SKV_LOC),
           pltpu.MemorySpace.HBM),
       pltpu.with_memory_space_constraint(
           jnp.transpose(V_ext, (0, 2, 3, 1)).reshape(B, HQ * DH, SKV_LOC),
           pltpu.MemorySpace.HBM),
       Wo)


# device time: 37067 ns/iter; 1.1591x vs baseline; 1.1101x over previous
import jax
import jax.numpy as jnp
from jax import lax
from jax.experimental import pallas as pl
from jax.experimental.pallas import tpu as pltpu

N_DEV = 8
B = 2
SQ = 256
DM = 512
DH = 64
HQ_LOC = 4
HQ = N_DEV * HQ_LOC
SKV_LOC = 256
BLK = 64
BF16 = jnp.bfloat16
SC = SQ // N_DEV

MESH = pl.DeviceIdType.MESH


def kernel(x, Wq, K_ext, V_ext, Wo):
    def body(x_ref, wq_ref, k_ref, v_ref, wo_ref, out_ref,
             xv, wqv, wov, kf32, vf32, kstage, vstage, kbuf, vbuf, rbufs,
             part, rsbuf, agstage, agbuf,
             copy_sems, ksend_sems, vsend_sems, krecv_sem, vrecv_sem,
             relay_recv_sems, fwd_send_sems,
             rssend_sems, rsrecv_sems, agsend_sems, agrecv_sems):
        me = lax.axis_index("i")

        def fetch(ref, dst, slot):
            return pltpu.make_async_copy(ref, dst, copy_sems.at[slot])

        def ph0_fetch():
            fetch(x_ref, xv, 2).start()
            fetch(wq_ref, wqv, 3).start()
            fetch(wo_ref, wov, 4).start()

        def slice_rdma(stage, d, dst, send_sem, recv_sem, target):
            return pltpu.make_async_remote_copy(
                src_ref=stage.at[:, pl.ds(HQ_LOC * DH * d, HQ_LOC * DH), :],
                dst_ref=dst, send_sem=send_sem, recv_sem=recv_sem,
                device_id=(target,), device_id_type=MESH)

        def kdir(d):
            return slice_rdma(kstage, d, kbuf, ksend_sems.at[d], krecv_sem, d)

        def vdir(d):
            return slice_rdma(vstage, d, vbuf, vsend_sems.at[d], vrecv_sem, d)

        def fwd(slot, dst, recv_sem, target):
            return pltpu.make_async_remote_copy(
                src_ref=rbufs.at[slot], dst_ref=dst,
                send_sem=fwd_send_sems.at[slot], recv_sem=recv_sem,
                device_id=(target,), device_id_type=MESH)

        def relay_wait(slot):
            pltpu.make_async_remote_copy(
                src_ref=rbufs.at[slot], dst_ref=rbufs.at[slot],
                send_sem=fwd_send_sems.at[slot],
                recv_sem=relay_recv_sems.at[slot],
                device_id=(0,), device_id_type=MESH).wait_recv()

        def ph1_bcast_issue():
            @pl.when(me == 0)
            def _():
                ck = pltpu.make_async_copy(k_ref, kf32, copy_sems.at[0])
                cv = pltpu.make_async_copy(v_ref, vf32, copy_sems.at[1])
                ck.start()
                cv.start()

                ck.wait()
                kstage[...] = kf32[...].astype(BF16)
                slice_rdma(kstage, 5, rbufs.at[0], ksend_sems.at[5],
                           relay_recv_sems.at[0], 4).start()
                slice_rdma(kstage, 6, rbufs.at[0], ksend_sems.at[6],
                           relay_recv_sems.at[0], 3).start()
                kdir(7).start()
                kdir(2).start()
                kdir(1).start()
                kbuf[...] = kstage[:, 0:HQ_LOC * DH, :]

                cv.wait()
                vstage[...] = vf32[...].astype(BF16)
                slice_rdma(vstage, 5, rbufs.at[1], vsend_sems.at[5],
                           relay_recv_sems.at[1], 4).start()
                slice_rdma(vstage, 7, rbufs.at[2], vsend_sems.at[7],
                           relay_recv_sems.at[2], 4).start()
                slice_rdma(vstage, 6, rbufs.at[1], vsend_sems.at[6],
                           relay_recv_sems.at[1], 3).start()
                vbuf[...] = vstage[:, 0:HQ_LOC * DH, :]

                kdir(2).wait_send()
                kdir(1).wait_send()
                vdir(2).start()
                vdir(1).start()

                slice_rdma(kstage, 5, rbufs.at[0], ksend_sems.at[5],
                           relay_recv_sems.at[0], 4).wait_send()
                slice_rdma(vstage, 5, rbufs.at[1], vsend_sems.at[5],
                           relay_recv_sems.at[1], 4).wait_send()
                slice_rdma(vstage, 7, rbufs.at[2], vsend_sems.at[7],
                           relay_recv_sems.at[2], 4).wait_send()
                kdir(4).start()
                vdir(4).start()

                slice_rdma(kstage, 6, rbufs.at[0], ksend_sems.at[6],
                           relay_recv_sems.at[0], 3).wait_send()
                slice_rdma(vstage, 6, rbufs.at[1], vsend_sems.at[6],
                           relay_recv_sems.at[1], 3).wait_send()
                kdir(7).wait_send()
                kdir(3).start()
                vdir(3).start()

        def ph1_relay():
            @pl.when(me == 4)
            def _():
                relay_wait(0)
                fwd(0, kbuf, krecv_sem, 5).start()
                relay_wait(1)
                fwd(1, vbuf, vrecv_sem, 5).start()
                relay_wait(2)
                fwd(2, vbuf, vrecv_sem, 7).start()

            @pl.when(me == 3)
            def _():
                relay_wait(0)
                fwd(0, kbuf, krecv_sem, 6).start()
                relay_wait(1)
                fwd(1, vbuf, vrecv_sem, 6).start()

        def ph2_qproj():
            fetch(x_ref, xv, 2).wait()
            fetch(wq_ref, wqv, 3).wait()
            wq = wqv[...].astype(BF16)
            qs = []
            for b in range(B):
                xb = xv[b].astype(BF16)
                q = jnp.dot(xb, wq, preferred_element_type=jnp.float32)
                qs.append((q * 0.125).astype(BF16))
            return qs

        def ph_k_wait():
            @pl.when(me != 0)
            def _():
                pltpu.make_async_remote_copy(
                    src_ref=kbuf, dst_ref=kbuf,
                    send_sem=ksend_sems.at[0], recv_sem=krecv_sem,
                    device_id=(0,), device_id_type=MESH).wait_recv()

        def ph_v_wait():
            @pl.when(me != 0)
            def _():
                pltpu.make_async_remote_copy(
                    src_ref=vbuf, dst_ref=vbuf,
                    send_sem=vsend_sems.at[0], recv_sem=vrecv_sem,
                    device_id=(0,), device_id_type=MESH).wait_recv()

        def ph3a_scores(qs):
            row = lax.broadcasted_iota(jnp.int32, (SQ, SKV_LOC), 0) // BLK
            col = lax.broadcasted_iota(jnp.int32, (SQ, SKV_LOC), 1) // BLK
            mask = col <= row
            ws = []
            for b in range(B):
                kb2 = kbuf[b]
                wb = []
                for h in range(HQ_LOC):
                    qh = qs[b][:, DH * h:DH * (h + 1)]
                    kh = kb2[DH * h:DH * (h + 1), :]
                    s = lax.dot_general(
                        qh, kh, (((1,), (0,)), ((), ())),
                        preferred_element_type=jnp.float32)
                    s = jnp.where(mask, s, -1e9)
                    mx = jnp.max(s, axis=1, keepdims=True)
                    w = jnp.exp(s - mx)
                    wb.append(
                        (w / jnp.sum(w, axis=1, keepdims=True)).astype(BF16))
                ws.append(wb)
            return ws

        def ph3b_ctx(ws):
            fetch(wo_ref, wov, 4).wait()
            wo = wov[...].astype(BF16)
            for b in range(B):
                vb2 = vbuf[b]
                acc = jnp.zeros((SQ, DM), jnp.float32)
                for h in range(HQ_LOC):
                    ch = lax.dot_general(
                        ws[b][h], vb2[DH * h:DH * (h + 1), :],
                        (((1,), (1,)), ((), ())),
                        preferred_element_type=jnp.float32)
                    acc = acc + jnp.dot(
                        ch.astype(BF16), wo[DH * h:DH * (h + 1), :],
                        preferred_element_type=jnp.float32)
                part[b] = acc.astype(BF16)

        def rs_rdma(s, d):
            return pltpu.make_async_remote_copy(
                src_ref=part.at[:, pl.ds(SC * d, SC), :],
                dst_ref=rsbuf.at[s],
                send_sem=rssend_sems.at[d], recv_sem=rsrecv_sems.at[s],
                device_id=(d,), device_id_type=MESH)

        def ag_rdma(s, d):
            return pltpu.make_async_remote_copy(
                src_ref=agstage, dst_ref=agbuf.at[s],
                send_sem=agsend_sems.at[d], recv_sem=agrecv_sems.at[s],
                device_id=(d,), device_id_type=MESH)

        def ph4_rs_issue():
            for s in range(N_DEV):
                @pl.when(me == s)
                def _(s=s):
                    rsbuf[s] = part[:, SC * s:SC * (s + 1), :]
                    for d in range(N_DEV):
                        if d != s:
                            rs_rdma(s, d).start()

        def ph4_rs_wait():
            for s in range(N_DEV):
                @pl.when(me != s)
                def _(s=s):
                    rs_rdma(s, 0).wait_recv()

        def ph4_reduce():
            red = rsbuf[0].astype(jnp.float32)
            for s in range(1, N_DEV):
                red = red + rsbuf[s].astype(jnp.float32)
            agstage[...] = red.astype(BF16)
            return red

        def ph4_ag_issue(red):
            for s in range(N_DEV):
                @pl.when(me == s)
                def _(s=s):
                    out_ref[:, SC * s:SC * (s + 1), :] = red
                    for d in range(N_DEV):
                        if d != s:
                            ag_rdma(s, d).start()
            for s in range(N_DEV):
                @pl.when(me == s)
                def _(s=s):
                    for d in range(N_DEV):
                        if d != s:
                            rs_rdma(s, d).wait_send()

        def ph4_ag_wait():
            for s in range(N_DEV):
                @pl.when(me != s)
                def _(s=s):
                    ag_rdma(s, 0).wait_recv()
                    out_ref[:, SC * s:SC * (s + 1), :] = (
                        agbuf[s].astype(jnp.float32))
            for s in range(N_DEV):
                @pl.when(me == s)
                def _(s=s):
                    for d in range(N_DEV):
                        if d != s:
                            ag_rdma(s, d).wait_send()

        def ph5_drain():
            @pl.when(me == 0)
            def _():
                for d in (1, 2, 3, 4):
                    vdir(d).wait_send()
                kdir(3).wait_send()
                kdir(4).wait_send()

            @pl.when(me == 4)
            def _():
                fwd(0, kbuf, krecv_sem, 5).wait_send()
                fwd(1, vbuf, vrecv_sem, 5).wait_send()
                fwd(2, vbuf, vrecv_sem, 7).wait_send()

            @pl.when(me == 3)
            def _():
                fwd(0, kbuf, krecv_sem, 6).wait_send()
                fwd(1, vbuf, vrecv_sem, 6).wait_send()

        with jax.named_scope("p0_fetch"):
            ph0_fetch()
        with jax.named_scope("p1_bcast_issue"):
            ph1_bcast_issue()
        with jax.named_scope("p1_relay"):
            ph1_relay()
        with jax.named_scope("p2_qproj"):
            qs = ph2_qproj()
        with jax.named_scope("p2_k_wait"):
            ph_k_wait()
        with jax.named_scope("p3_scores"):
            ws = ph3a_scores(qs)
        with jax.named_scope("p2_v_wait"):
            ph_v_wait()
        with jax.named_scope("p3_ctx"):
            ph3b_ctx(ws)
        with jax.named_scope("p4_rs_issue"):
            ph4_rs_issue()
        with jax.named_scope("p4_rs_wait"):
            ph4_rs_wait()
        with jax.named_scope("p4_reduce"):
            red = ph4_reduce()
        with jax.named_scope("p4_ag_issue"):
            ph4_ag_issue(red)
        with jax.named_scope("p4_ag_wait"):
            ph4_ag_wait()
        with jax.named_scope("p5_drain"):
            ph5_drain()

    return pl.pallas_call(
        body,
        out_shape=jax.ShapeDtypeStruct((B, SQ, DM), jnp.float32),
        in_specs=[pl.BlockSpec(memory_space=pl.ANY)] * 5,
        out_specs=pl.BlockSpec(memory_space=pltpu.VMEM),
        scratch_shapes=[
            pltpu.VMEM((B, SQ, DM), jnp.float32),
            pltpu.VMEM((DM, HQ_LOC * DH), jnp.float32),
            pltpu.VMEM((HQ_LOC * DH, DM), jnp.float32),
            pltpu.VMEM((B, HQ * DH, SKV_LOC), jnp.float32),
            pltpu.VMEM((B, HQ * DH, SKV_LOC), jnp.float32),
            pltpu.VMEM((B, HQ * DH, SKV_LOC), BF16),
            pltpu.VMEM((B, HQ * DH, SKV_LOC), BF16),
            pltpu.VMEM((B, HQ_LOC * DH, SKV_LOC), BF16),
            pltpu.VMEM((B, HQ_LOC * DH, SKV_LOC), BF16),
            pltpu.VMEM((3, B, HQ_LOC * DH, SKV_LOC), BF16),
            pltpu.VMEM((B, SQ, DM), BF16),
            pltpu.VMEM((N_DEV, B, SC, DM), BF16),
            pltpu.VMEM((B, SC, DM), BF16),
            pltpu.VMEM((N_DEV, B, SC, DM), BF16),
            pltpu.SemaphoreType.DMA((5,)),
            pltpu.SemaphoreType.DMA((N_DEV,)),
            pltpu.SemaphoreType.DMA((N_DEV,)),
            pltpu.SemaphoreType.DMA,
            pltpu.SemaphoreType.DMA,
            pltpu.SemaphoreType.DMA((3,)),
            pltpu.SemaphoreType.DMA((3,)),
            pltpu.SemaphoreType.DMA((N_DEV,)),
            pltpu.SemaphoreType.DMA((N_DEV,)),
            pltpu.SemaphoreType.DMA((N_DEV,)),
            pltpu.SemaphoreType.DMA((N_DEV,)),
        ],
    )(pltpu.with_memory_space_constraint(x, pltpu.MemorySpace.HBM),
       pltpu.with_memory_space_constraint(Wq, pltpu.MemorySpace.HBM),
       pltpu.with_memory_space_constraint(
           jnp.transpose(K_ext, (0, 2, 3, 1)).reshape(B, HQ * DH, SKV_LOC),
           pltpu.MemorySpace.HBM),
       pltpu.with_memory_space_constraint(
           jnp.transpose(V_ext, (0, 2, 3, 1)).reshape(B, HQ * DH, SKV_LOC),
           pltpu.MemorySpace.HBM),
       pltpu.with_memory_space_constraint(Wo, pltpu.MemorySpace.HBM))
